# Initial kernel scaffold; baseline (speedup 1.0000x reference)
import functools

import jax
import jax.numpy as jnp
from jax import lax
from jax.experimental import pallas as pl
from jax.experimental.pallas import tpu as pltpu
from jax.experimental.pallas import tpu_sc as plsc

N = 50000
E = 800000
D_IN = 1433
H1 = 64
H2 = 32

NC = 2
NS = 16
LANES = 16

CH = 128
NCHUNK = E // CH
ROWS_PER_TILE = N // NS
ZR = 625

_mesh = plsc.VectorSubcoreMesh(
    core_axis_name="c", subcore_axis_name="s", num_cores=NC, num_subcores=NS
)


def _zero_stage(stage_ref, width):
    def body(i, _):
        stage_ref[i, :] = jnp.zeros((LANES,), jnp.float32)
        return 0
    lax.fori_loop(0, ZR * (width // LANES), body, 0, unroll=4)


def _deg_body(dst_hbm, out_hbm, deg_sh, idx_v, ones_v, stage_v):
    c = lax.axis_index("c")
    s = lax.axis_index("s")

    def fill_ones(i, _):
        ones_v[i, :] = jnp.ones((LANES,), jnp.float32)
        return 0
    lax.fori_loop(0, CH, fill_ones, 0, unroll=4)
    _zero_stage(stage_v, LANES)

    r0 = s * ROWS_PER_TILE

    def zero_acc(j, _):
        pltpu.sync_copy(stage_v, deg_sh.at[pl.ds(r0 + j * ZR, ZR)])
        return 0
    lax.fori_loop(0, ROWS_PER_TILE // ZR, zero_acc, 0)
    plsc.subcore_barrier()

    half = NCHUNK // NC
    base_chunks = half // NS
    nch = base_chunks + jnp.where(s < half - base_chunks * NS, 1, 0)

    def edge_loop(k, _):
        j = c * half + k * NS + s
        pltpu.sync_copy(dst_hbm.at[pl.ds(j * CH, CH)], idx_v)
        pltpu.sync_copy(ones_v, deg_sh.at[idx_v], add=True)
        return 0
    lax.fori_loop(0, nch, edge_loop, 0)
    plsc.subcore_barrier()

    def out_loop(j, _):
        rr = r0 + j * ZR
        pltpu.sync_copy(deg_sh.at[pl.ds(rr, ZR)], stage_v)
        pltpu.sync_copy(stage_v, out_hbm.at[pl.ds(rr, ZR), pl.ds(c * LANES, LANES)])
        return 0
    lax.fori_loop(0, ROWS_PER_TILE // ZR, out_loop, 0)


_deg_kernel = functools.partial(
    pl.kernel,
    out_type=jax.ShapeDtypeStruct((N, 2 * LANES), jnp.float32),
    mesh=_mesh,
    scratch_types=[
        pltpu.VMEM_SHARED((N, LANES), jnp.float32),
        pltpu.VMEM((CH,), jnp.int32),
        pltpu.VMEM((CH, LANES), jnp.float32),
        pltpu.VMEM((ZR, LANES), jnp.float32),
    ],
)(_deg_body)


def _agg_body(dh, tA_hbm, tB_hbm, src_hbm, dst_hbm, outA_hbm, outB_hbm,
              acc_sh, idx_s, idx_d, rows_v, stage_v, sem):
    c = lax.axis_index("c")
    s = lax.axis_index("s")

    _zero_stage(stage_v, dh)
    r0 = s * ROWS_PER_TILE

    def zero_acc(j, _):
        pltpu.sync_copy(stage_v, acc_sh.at[pl.ds(r0 + j * ZR, ZR)])
        return 0
    lax.fori_loop(0, ROWS_PER_TILE // ZR, zero_acc, 0)
    plsc.subcore_barrier()

    base_chunks = NCHUNK // NS
    nch = base_chunks + jnp.where(s < NCHUNK - base_chunks * NS, 1, 0)

    def edge_loop(k, _):
        j = k * NS + s
        pltpu.sync_copy(src_hbm.at[pl.ds(j * CH, CH)], idx_s)
        pltpu.sync_copy(dst_hbm.at[pl.ds(j * CH, CH)], idx_d)

        @pl.when(c == 0)
        def _():
            pltpu.async_copy(tA_hbm.at[idx_s], rows_v, sem).wait()

        @pl.when(c == 1)
        def _():
            pltpu.async_copy(tB_hbm.at[idx_s], rows_v, sem).wait()

        pltpu.sync_copy(rows_v, acc_sh.at[idx_d], add=True)
        return 0
    lax.fori_loop(0, nch, edge_loop, 0)
    plsc.subcore_barrier()

    def out_loop(j, _):
        rr = r0 + j * ZR
        pltpu.sync_copy(acc_sh.at[pl.ds(rr, ZR)], stage_v)

        @pl.when(c == 0)
        def _():
            pltpu.sync_copy(stage_v, outA_hbm.at[pl.ds(rr, ZR)])

        @pl.when(c == 1)
        def _():
            pltpu.sync_copy(stage_v, outB_hbm.at[pl.ds(rr, ZR)])
        return 0
    lax.fori_loop(0, ROWS_PER_TILE // ZR, out_loop, 0)


def _make_agg(dh):
    return functools.partial(
        pl.kernel,
        out_type=[
            jax.ShapeDtypeStruct((N, dh), jnp.float32),
            jax.ShapeDtypeStruct((N, dh), jnp.float32),
        ],
        mesh=_mesh,
        scratch_types=[
            pltpu.VMEM_SHARED((N, dh), jnp.float32),
            pltpu.VMEM((CH,), jnp.int32),
            pltpu.VMEM((CH,), jnp.int32),
            pltpu.VMEM((CH, dh), jnp.float32),
            pltpu.VMEM((ZR, dh), jnp.float32),
            pltpu.SemaphoreType.DMA,
        ],
    )(functools.partial(_agg_body, dh))


_agg1 = _make_agg(H1 // 2)
_agg2 = _make_agg(H2 // 2)


BN_MM = 400
BN_EP = 1000


def _mm1_body(x_ref, w_ref, hA_ref, hB_ref, xr_ref):
    g = jnp.dot(x_ref[...], w_ref[...], preferred_element_type=jnp.float32)
    hA_ref[...] = g[:, 0:H1 // 2]
    hB_ref[...] = g[:, H1 // 2:H1]
    xr_ref[...] = g[:, H1:]


def _mm1(x, w1cat):
    return pl.pallas_call(
        _mm1_body,
        grid=(N // BN_MM,),
        in_specs=[
            pl.BlockSpec((BN_MM, D_IN), lambda i: (i, 0)),
            pl.BlockSpec((D_IN, 2 * H1), lambda i: (0, 0)),
        ],
        out_specs=[
            pl.BlockSpec((BN_MM, H1 // 2), lambda i: (i, 0)),
            pl.BlockSpec((BN_MM, H1 // 2), lambda i: (i, 0)),
            pl.BlockSpec((BN_MM, H1), lambda i: (i, 0)),
        ],
        out_shape=[
            jax.ShapeDtypeStruct((N, H1 // 2), jnp.float32),
            jax.ShapeDtypeStruct((N, H1 // 2), jnp.float32),
            jax.ShapeDtypeStruct((N, H1), jnp.float32),
        ],
    )(x, w1cat)


def _ep1_body(accA_ref, accB_ref, degp_ref, xr_ref, w2_ref, b1_ref,
              gA_ref, gB_ref, gr_ref, recip_ref):
    deg = degp_ref[:, 0:1] + degp_ref[:, LANES:LANES + 1]
    recip = 1.0 / jnp.maximum(deg, 1.0)
    summed = jnp.concatenate([accA_ref[...], accB_ref[...]], axis=1)
    h1 = summed * recip + b1_ref[...] + xr_ref[...]
    g = jnp.dot(h1, w2_ref[...], preferred_element_type=jnp.float32)
    gA_ref[...] = g[:, 0:H2 // 2]
    gB_ref[...] = g[:, H2 // 2:H2]
    gr_ref[...] = g[:, H2:]
    recip_ref[...] = recip


def _ep1(accA, accB, degp, xr, w2cat, b1r):
    return pl.pallas_call(
        _ep1_body,
        grid=(N // BN_EP,),
        in_specs=[
            pl.BlockSpec((BN_EP, H1 // 2), lambda i: (i, 0)),
            pl.BlockSpec((BN_EP, H1 // 2), lambda i: (i, 0)),
            pl.BlockSpec((BN_EP, 2 * LANES), lambda i: (i, 0)),
            pl.BlockSpec((BN_EP, H1), lambda i: (i, 0)),
            pl.BlockSpec((H1, H1), lambda i: (0, 0)),
            pl.BlockSpec((1, H1), lambda i: (0, 0)),
        ],
        out_specs=[
            pl.BlockSpec((BN_EP, H2 // 2), lambda i: (i, 0)),
            pl.BlockSpec((BN_EP, H2 // 2), lambda i: (i, 0)),
            pl.BlockSpec((BN_EP, H2), lambda i: (i, 0)),
            pl.BlockSpec((BN_EP, 1), lambda i: (i, 0)),
        ],
        out_shape=[
            jax.ShapeDtypeStruct((N, H2 // 2), jnp.float32),
            jax.ShapeDtypeStruct((N, H2 // 2), jnp.float32),
            jax.ShapeDtypeStruct((N, H2), jnp.float32),
            jax.ShapeDtypeStruct((N, 1), jnp.float32),
        ],
    )(accA, accB, degp, xr, w2cat, b1r)


def _ep2_body(accA_ref, accB_ref, recip_ref, gr_ref, b2_ref, out_ref):
    summed = jnp.concatenate([accA_ref[...], accB_ref[...]], axis=1)
    h2 = summed * recip_ref[...] + b2_ref[...] + gr_ref[...]
    y = jnp.maximum(h2, 0.0)
    m = jnp.max(y, axis=1, keepdims=True)
    lse = jnp.log(jnp.sum(jnp.exp(y - m), axis=1, keepdims=True)) + m
    out_ref[...] = y - lse


def _ep2(accA, accB, recip, gr, b2r):
    return pl.pallas_call(
        _ep2_body,
        grid=(N // BN_EP,),
        in_specs=[
            pl.BlockSpec((BN_EP, H2 // 2), lambda i: (i, 0)),
            pl.BlockSpec((BN_EP, H2 // 2), lambda i: (i, 0)),
            pl.BlockSpec((BN_EP, 1), lambda i: (i, 0)),
            pl.BlockSpec((BN_EP, H2), lambda i: (i, 0)),
            pl.BlockSpec((1, H2), lambda i: (0, 0)),
        ],
        out_specs=pl.BlockSpec((BN_EP, H2), lambda i: (i, 0)),
        out_shape=jax.ShapeDtypeStruct((N, H2), jnp.float32),
    )(accA, accB, recip, gr, b2r)


def kernel(x, edge_index, Wl1, Wr1, b1, Wl2, Wr2, b2):
    src = edge_index[0]
    dst = edge_index[1]
    w1cat = jnp.concatenate([Wl1, Wr1], axis=1)
    w2cat = jnp.concatenate([Wl2, Wr2], axis=1)
    b1r = b1.reshape(1, H1)
    b2r = b2.reshape(1, H2)

    hA, hB, xr = _mm1(x, w1cat)
    degp = _deg_kernel(dst)
    accA, accB = _agg1(hA, hB, src, dst)
    gA, gB, gr, recip = _ep1(accA, accB, degp, xr, w2cat, b1r)
    acc2A, acc2B = _agg2(gA, gB, src, dst)
    return _ep2(acc2A, acc2B, recip, gr, b2r)


# speedup vs baseline: 3.3811x; 3.3811x over previous
import functools

import jax
import jax.numpy as jnp
from jax import lax
from jax.experimental import pallas as pl
from jax.experimental.pallas import tpu as pltpu
from jax.experimental.pallas import tpu_sc as plsc

N = 50000
E = 800000
D_IN = 1433
H1 = 64
H2 = 32

NC = 2
NS = 16
LANES = 16

CH = 128
NCHUNK = E // CH
NPAD = 50176
ROWS_PER_TILE = NPAD // NS
ZR = 784

_mesh = plsc.VectorSubcoreMesh(
    core_axis_name="c", subcore_axis_name="s", num_cores=NC, num_subcores=NS
)


def _zero_stage(stage_ref, width):
    nseg = width // LANES

    def body(i, _):
        for j in range(nseg):
            stage_ref[i, pl.ds(j * LANES, LANES)] = jnp.zeros((LANES,), jnp.float32)
        return 0
    lax.fori_loop(0, ZR, body, 0, unroll=4)


def _deg_body(dst_hbm, out0_hbm, out1_hbm, deg_sh, idx_v, ones_v, stage_v):
    c = lax.axis_index("c")
    s = lax.axis_index("s")

    def fill_ones(i, _):
        ones_v[i, :] = jnp.ones((LANES,), jnp.float32)
        return 0
    lax.fori_loop(0, CH, fill_ones, 0, unroll=4)
    _zero_stage(stage_v, LANES)

    r0 = s * ROWS_PER_TILE

    def zero_acc(j, _):
        pltpu.sync_copy(stage_v, deg_sh.at[pl.ds(r0 + j * ZR, ZR)])
        return 0
    lax.fori_loop(0, ROWS_PER_TILE // ZR, zero_acc, 0)
    plsc.subcore_barrier()

    half = NCHUNK // NC
    base_chunks = half // NS
    nch = base_chunks + jnp.where(s < half - base_chunks * NS, 1, 0)

    def edge_loop(k, _):
        j = c * half + k * NS + s
        pltpu.sync_copy(dst_hbm.at[pl.ds(j * CH, CH)], idx_v)
        pltpu.sync_copy(ones_v, deg_sh.at[idx_v], add=True)
        return 0
    lax.fori_loop(0, nch, edge_loop, 0)
    plsc.subcore_barrier()

    def out_loop(j, _):
        rr = r0 + j * ZR
        pltpu.sync_copy(deg_sh.at[pl.ds(rr, ZR)], stage_v)

        @pl.when(c == 0)
        def _():
            pltpu.sync_copy(stage_v, out0_hbm.at[pl.ds(rr, ZR)])

        @pl.when(c == 1)
        def _():
            pltpu.sync_copy(stage_v, out1_hbm.at[pl.ds(rr, ZR)])
        return 0
    lax.fori_loop(0, ROWS_PER_TILE // ZR, out_loop, 0)


_deg_kernel = functools.partial(
    pl.kernel,
    out_type=[
        jax.ShapeDtypeStruct((NPAD, LANES), jnp.float32),
        jax.ShapeDtypeStruct((NPAD, LANES), jnp.float32),
    ],
    mesh=_mesh,
    compiler_params=pltpu.CompilerParams(use_tc_tiling_on_sc=False),
    scratch_types=[
        pltpu.VMEM_SHARED((NPAD, LANES), jnp.float32),
        pltpu.VMEM((CH,), jnp.int32),
        pltpu.VMEM((CH, LANES), jnp.float32),
        pltpu.VMEM((ZR, LANES), jnp.float32),
    ],
)(_deg_body)


def _agg_body(dh, tA_hbm, tB_hbm, src_hbm, dst_hbm, outA_hbm, outB_hbm,
              acc_sh, idx_s, idx_d, rows_v, stage_v, sem):
    c = lax.axis_index("c")
    s = lax.axis_index("s")

    _zero_stage(stage_v, dh)
    r0 = s * ROWS_PER_TILE

    def zero_acc(j, _):
        pltpu.sync_copy(stage_v, acc_sh.at[pl.ds(r0 + j * ZR, ZR)])
        return 0
    lax.fori_loop(0, ROWS_PER_TILE // ZR, zero_acc, 0)
    plsc.subcore_barrier()

    base_chunks = NCHUNK // NS
    nch = base_chunks + jnp.where(s < NCHUNK - base_chunks * NS, 1, 0)

    def edge_loop(k, _):
        j = k * NS + s
        pltpu.sync_copy(src_hbm.at[pl.ds(j * CH, CH)], idx_s)
        pltpu.sync_copy(dst_hbm.at[pl.ds(j * CH, CH)], idx_d)

        @pl.when(c == 0)
        def _():
            pltpu.async_copy(tA_hbm.at[idx_s], rows_v, sem).wait()

        @pl.when(c == 1)
        def _():
            pltpu.async_copy(tB_hbm.at[idx_s], rows_v, sem).wait()

        pltpu.sync_copy(rows_v, acc_sh.at[idx_d], add=True)
        return 0
    lax.fori_loop(0, nch, edge_loop, 0)
    plsc.subcore_barrier()

    def out_loop(j, _):
        rr = r0 + j * ZR
        pltpu.sync_copy(acc_sh.at[pl.ds(rr, ZR)], stage_v)

        @pl.when(c == 0)
        def _():
            pltpu.sync_copy(stage_v, outA_hbm.at[pl.ds(rr, ZR)])

        @pl.when(c == 1)
        def _():
            pltpu.sync_copy(stage_v, outB_hbm.at[pl.ds(rr, ZR)])
        return 0
    lax.fori_loop(0, ROWS_PER_TILE // ZR, out_loop, 0)


def _make_agg(dh):
    return functools.partial(
        pl.kernel,
        out_type=[
            jax.ShapeDtypeStruct((NPAD, dh), jnp.float32),
            jax.ShapeDtypeStruct((NPAD, dh), jnp.float32),
        ],
        mesh=_mesh,
        compiler_params=pltpu.CompilerParams(use_tc_tiling_on_sc=False),
        scratch_types=[
            pltpu.VMEM_SHARED((NPAD, dh), jnp.float32),
            pltpu.VMEM((CH,), jnp.int32),
            pltpu.VMEM((CH,), jnp.int32),
            pltpu.VMEM((CH, dh), jnp.float32),
            pltpu.VMEM((ZR, dh), jnp.float32),
            pltpu.SemaphoreType.DMA,
        ],
    )(functools.partial(_agg_body, dh))


_agg1 = _make_agg(H1 // 2)
_agg2 = _make_agg(H2 // 2)


BN_MM = 400
BN_EP = 1000


def _mm1_body(x_ref, w_ref, hA_ref, hB_ref, xr_ref):
    g = jnp.dot(x_ref[...], w_ref[...], preferred_element_type=jnp.float32)
    hA_ref[...] = g[:, 0:H1 // 2]
    hB_ref[...] = g[:, H1 // 2:H1]
    xr_ref[...] = g[:, H1:]


def _mm1(x, w1cat):
    return pl.pallas_call(
        _mm1_body,
        grid=(N // BN_MM,),
        in_specs=[
            pl.BlockSpec((BN_MM, D_IN), lambda i: (i, 0)),
            pl.BlockSpec((D_IN, 2 * H1), lambda i: (0, 0)),
        ],
        out_specs=[
            pl.BlockSpec((BN_MM, H1 // 2), lambda i: (i, 0)),
            pl.BlockSpec((BN_MM, H1 // 2), lambda i: (i, 0)),
            pl.BlockSpec((BN_MM, H1), lambda i: (i, 0)),
        ],
        out_shape=[
            jax.ShapeDtypeStruct((N, H1 // 2), jnp.float32),
            jax.ShapeDtypeStruct((N, H1 // 2), jnp.float32),
            jax.ShapeDtypeStruct((N, H1), jnp.float32),
        ],
    )(x, w1cat)


def _ep1_body(accA_ref, accB_ref, deg0_ref, deg1_ref, xr_ref, w2_ref, b1_ref,
              gA_ref, gB_ref, gr_ref, recip_ref):
    deg = deg0_ref[:, 0:1] + deg1_ref[:, 0:1]
    recip = 1.0 / jnp.maximum(deg, 1.0)
    summed = jnp.concatenate([accA_ref[...], accB_ref[...]], axis=1)
    h1 = summed * recip + b1_ref[...] + xr_ref[...]
    g = jnp.dot(h1, w2_ref[...], preferred_element_type=jnp.float32)
    gA_ref[...] = g[:, 0:H2 // 2]
    gB_ref[...] = g[:, H2 // 2:H2]
    gr_ref[...] = g[:, H2:]
    recip_ref[...] = recip


def _ep1(accA, accB, deg0, deg1, xr, w2cat, b1r):
    return pl.pallas_call(
        _ep1_body,
        grid=(N // BN_EP,),
        in_specs=[
            pl.BlockSpec((BN_EP, H1 // 2), lambda i: (i, 0)),
            pl.BlockSpec((BN_EP, H1 // 2), lambda i: (i, 0)),
            pl.BlockSpec((BN_EP, LANES), lambda i: (i, 0)),
            pl.BlockSpec((BN_EP, LANES), lambda i: (i, 0)),
            pl.BlockSpec((BN_EP, H1), lambda i: (i, 0)),
            pl.BlockSpec((H1, H1), lambda i: (0, 0)),
            pl.BlockSpec((1, H1), lambda i: (0, 0)),
        ],
        out_specs=[
            pl.BlockSpec((BN_EP, H2 // 2), lambda i: (i, 0)),
            pl.BlockSpec((BN_EP, H2 // 2), lambda i: (i, 0)),
            pl.BlockSpec((BN_EP, H2), lambda i: (i, 0)),
            pl.BlockSpec((BN_EP, 1), lambda i: (i, 0)),
        ],
        out_shape=[
            jax.ShapeDtypeStruct((N, H2 // 2), jnp.float32),
            jax.ShapeDtypeStruct((N, H2 // 2), jnp.float32),
            jax.ShapeDtypeStruct((N, H2), jnp.float32),
            jax.ShapeDtypeStruct((N, 1), jnp.float32),
        ],
    )(accA, accB, deg0, deg1, xr, w2cat, b1r)


def _ep2_body(accA_ref, accB_ref, recip_ref, gr_ref, b2_ref, out_ref):
    summed = jnp.concatenate([accA_ref[...], accB_ref[...]], axis=1)
    h2 = summed * recip_ref[...] + b2_ref[...] + gr_ref[...]
    y = jnp.maximum(h2, 0.0)
    m = jnp.max(y, axis=1, keepdims=True)
    lse = jnp.log(jnp.sum(jnp.exp(y - m), axis=1, keepdims=True)) + m
    out_ref[...] = y - lse


def _ep2(accA, accB, recip, gr, b2r):
    return pl.pallas_call(
        _ep2_body,
        grid=(N // BN_EP,),
        in_specs=[
            pl.BlockSpec((BN_EP, H2 // 2), lambda i: (i, 0)),
            pl.BlockSpec((BN_EP, H2 // 2), lambda i: (i, 0)),
            pl.BlockSpec((BN_EP, 1), lambda i: (i, 0)),
            pl.BlockSpec((BN_EP, H2), lambda i: (i, 0)),
            pl.BlockSpec((1, H2), lambda i: (0, 0)),
        ],
        out_specs=pl.BlockSpec((BN_EP, H2), lambda i: (i, 0)),
        out_shape=jax.ShapeDtypeStruct((N, H2), jnp.float32),
    )(accA, accB, recip, gr, b2r)


def kernel(x, edge_index, Wl1, Wr1, b1, Wl2, Wr2, b2):
    src = edge_index[0]
    dst = edge_index[1]
    w1cat = jnp.concatenate([Wl1, Wr1], axis=1)
    w2cat = jnp.concatenate([Wl2, Wr2], axis=1)
    b1r = b1.reshape(1, H1)
    b2r = b2.reshape(1, H2)

    hA, hB, xr = _mm1(x, w1cat)
    deg0, deg1 = _deg_kernel(dst)
    accA, accB = _agg1(hA, hB, src, dst)
    gA, gB, gr, recip = _ep1(accA, accB, deg0, deg1, xr, w2cat, b1r)
    acc2A, acc2B = _agg2(gA, gB, src, dst)
    return _ep2(acc2A, acc2B, recip, gr, b2r)


# speedup vs baseline: 6.2243x; 1.8409x over previous
import functools

import jax
import jax.numpy as jnp
from jax import lax
from jax.experimental import pallas as pl
from jax.experimental.pallas import tpu as pltpu
from jax.experimental.pallas import tpu_sc as plsc

N = 50000
E = 800000
D_IN = 1433
H1 = 64
H2 = 32

NC = 2
NS = 16
LANES = 16

CH = 128
EPAD = 802816
ECH = EPAD // CH
KB1 = 4
KB2 = 8
KD = 4
NBATCH_D = ECH // NC // NS // KD
NPAD = 50176
ROWS_PER_TILE = NPAD // NS
ZR = 392

_mesh = plsc.VectorSubcoreMesh(
    core_axis_name="c", subcore_axis_name="s", num_cores=NC, num_subcores=NS
)


def _zero_stage(stage_ref, width):
    nseg = width // LANES
    nrows = stage_ref.shape[0]

    def body(i, _):
        for j in range(nseg):
            stage_ref[i, pl.ds(j * LANES, LANES)] = jnp.zeros((LANES,), jnp.float32)
        return 0
    lax.fori_loop(0, nrows, body, 0, unroll=4)


def _deg_body(dst_hbm, out0_hbm, out1_hbm, deg_sh, idx_d2, ones_v, stage_v, sem):
    c = lax.axis_index("c")
    s = lax.axis_index("s")

    def fill_ones(i, _):
        ones_v[i, :] = jnp.ones((LANES,), jnp.float32)
        return 0
    lax.fori_loop(0, CH, fill_ones, 0, unroll=4)
    _zero_stage(stage_v, LANES)

    r0 = s * ROWS_PER_TILE

    def zero_acc(j, _):
        pltpu.sync_copy(stage_v, deg_sh.at[pl.ds(r0 + j * ZR, ZR)])
        return 0
    lax.fori_loop(0, ROWS_PER_TILE // ZR, zero_acc, 0)
    plsc.subcore_barrier()

    base = (c * NS + s) * (ECH // NC // NS)

    def edge_loop(b, _):
        r = base + b * KD
        pltpu.async_copy(dst_hbm.at[pl.ds(r, KD)], idx_d2, sem).wait()
        descs = [
            pltpu.async_copy(ones_v, deg_sh.at[idx_d2.at[j]], sem, add=True)
            for j in range(KD)
        ]
        for d in descs:
            d.wait()
        return 0
    lax.fori_loop(0, NBATCH_D, edge_loop, 0)
    plsc.subcore_barrier()

    def out_loop(j, _):
        rr = r0 + j * ZR
        pltpu.sync_copy(deg_sh.at[pl.ds(rr, ZR)], stage_v)

        @pl.when(c == 0)
        def _():
            pltpu.sync_copy(stage_v, out0_hbm.at[pl.ds(rr, ZR)])

        @pl.when(c == 1)
        def _():
            pltpu.sync_copy(stage_v, out1_hbm.at[pl.ds(rr, ZR)])
        return 0
    lax.fori_loop(0, ROWS_PER_TILE // ZR, out_loop, 0)


_deg_kernel = functools.partial(
    pl.kernel,
    out_type=[
        jax.ShapeDtypeStruct((NPAD, LANES), jnp.float32),
        jax.ShapeDtypeStruct((NPAD, LANES), jnp.float32),
    ],
    mesh=_mesh,
    compiler_params=pltpu.CompilerParams(use_tc_tiling_on_sc=False),
    scratch_types=[
        pltpu.VMEM_SHARED((NPAD, LANES), jnp.float32),
        pltpu.VMEM((KD, CH), jnp.int32),
        pltpu.VMEM((CH, LANES), jnp.float32),
        pltpu.VMEM((ZR, LANES), jnp.float32),
        pltpu.SemaphoreType.DMA,
    ],
)(_deg_body)


def _agg_body(dh, kb, tA_hbm, tB_hbm, src_hbm, dst_hbm, outA_hbm, outB_hbm,
              acc_sh, idx_s2, idx_d2, rows2, sem):
    c = lax.axis_index("c")
    s = lax.axis_index("s")

    _zero_stage(rows2, dh)
    r0 = s * ROWS_PER_TILE

    def zero_acc(j, _):
        pltpu.sync_copy(rows2.at[pl.ds(0, ZR)], acc_sh.at[pl.ds(r0 + j * ZR, ZR)])
        return 0
    lax.fori_loop(0, ROWS_PER_TILE // ZR, zero_acc, 0)
    plsc.subcore_barrier()

    base = s * (ECH // NS)
    nbatch = ECH // NS // kb

    def edge_loop(b, _):
        r = base + b * kb
        di = pltpu.async_copy(src_hbm.at[pl.ds(r, kb)], idx_s2, sem)
        dj = pltpu.async_copy(dst_hbm.at[pl.ds(r, kb)], idx_d2, sem)
        di.wait()
        dj.wait()

        @pl.when(c == 0)
        def _():
            descs = [
                pltpu.async_copy(
                    tA_hbm.at[idx_s2.at[j]],
                    rows2.at[pl.ds(j * CH, CH)], sem)
                for j in range(kb)
            ]
            for d in descs:
                d.wait()

        @pl.when(c == 1)
        def _():
            descs = [
                pltpu.async_copy(
                    tB_hbm.at[idx_s2.at[j]],
                    rows2.at[pl.ds(j * CH, CH)], sem)
                for j in range(kb)
            ]
            for d in descs:
                d.wait()

        descs = [
            pltpu.async_copy(
                rows2.at[pl.ds(j * CH, CH)],
                acc_sh.at[idx_d2.at[j]], sem, add=True)
            for j in range(kb)
        ]
        for d in descs:
            d.wait()
        return 0
    lax.fori_loop(0, nbatch, edge_loop, 0)
    plsc.subcore_barrier()

    def out_loop(j, _):
        rr = r0 + j * ZR
        pltpu.sync_copy(acc_sh.at[pl.ds(rr, ZR)], rows2.at[pl.ds(0, ZR)])

        @pl.when(c == 0)
        def _():
            pltpu.sync_copy(rows2.at[pl.ds(0, ZR)], outA_hbm.at[pl.ds(rr, ZR)])

        @pl.when(c == 1)
        def _():
            pltpu.sync_copy(rows2.at[pl.ds(0, ZR)], outB_hbm.at[pl.ds(rr, ZR)])
        return 0
    lax.fori_loop(0, ROWS_PER_TILE // ZR, out_loop, 0)


def _make_agg(dh, kb):
    return functools.partial(
        pl.kernel,
        out_type=[
            jax.ShapeDtypeStruct((NPAD, dh), jnp.float32),
            jax.ShapeDtypeStruct((NPAD, dh), jnp.float32),
        ],
        mesh=_mesh,
        compiler_params=pltpu.CompilerParams(use_tc_tiling_on_sc=False),
        scratch_types=[
            pltpu.VMEM_SHARED((NPAD, dh), jnp.float32),
            pltpu.VMEM((kb, CH), jnp.int32),
            pltpu.VMEM((kb, CH), jnp.int32),
            pltpu.VMEM((kb * CH, dh), jnp.float32),
            pltpu.SemaphoreType.DMA,
        ],
    )(functools.partial(_agg_body, dh, kb))


_agg1 = _make_agg(H1 // 2, KB1)
_agg2 = _make_agg(H2 // 2, KB2)


BN_MM = 400
BN_EP = 1000


def _mm1_body(x_ref, w_ref, hA_ref, hB_ref, xr_ref):
    g = jnp.dot(x_ref[...], w_ref[...], preferred_element_type=jnp.float32)
    hA_ref[...] = g[:, 0:H1 // 2]
    hB_ref[...] = g[:, H1 // 2:H1]
    xr_ref[...] = g[:, H1:]


def _mm1(x, w1cat):
    return pl.pallas_call(
        _mm1_body,
        grid=(N // BN_MM,),
        in_specs=[
            pl.BlockSpec((BN_MM, D_IN), lambda i: (i, 0)),
            pl.BlockSpec((D_IN, 2 * H1), lambda i: (0, 0)),
        ],
        out_specs=[
            pl.BlockSpec((BN_MM, H1 // 2), lambda i: (i, 0)),
            pl.BlockSpec((BN_MM, H1 // 2), lambda i: (i, 0)),
            pl.BlockSpec((BN_MM, H1), lambda i: (i, 0)),
        ],
        out_shape=[
            jax.ShapeDtypeStruct((N, H1 // 2), jnp.float32),
            jax.ShapeDtypeStruct((N, H1 // 2), jnp.float32),
            jax.ShapeDtypeStruct((N, H1), jnp.float32),
        ],
    )(x, w1cat)


def _ep1_body(accA_ref, accB_ref, deg0_ref, deg1_ref, xr_ref, w2_ref, b1_ref,
              gA_ref, gB_ref, gr_ref, recip_ref):
    deg = deg0_ref[:, 0:1] + deg1_ref[:, 0:1]
    recip = 1.0 / jnp.maximum(deg, 1.0)
    summed = jnp.concatenate([accA_ref[...], accB_ref[...]], axis=1)
    h1 = summed * recip + b1_ref[...] + xr_ref[...]
    g = jnp.dot(h1, w2_ref[...], preferred_element_type=jnp.float32)
    gA_ref[...] = g[:, 0:H2 // 2]
    gB_ref[...] = g[:, H2 // 2:H2]
    gr_ref[...] = g[:, H2:]
    recip_ref[...] = recip


def _ep1(accA, accB, deg0, deg1, xr, w2cat, b1r):
    return pl.pallas_call(
        _ep1_body,
        grid=(N // BN_EP,),
        in_specs=[
            pl.BlockSpec((BN_EP, H1 // 2), lambda i: (i, 0)),
            pl.BlockSpec((BN_EP, H1 // 2), lambda i: (i, 0)),
            pl.BlockSpec((BN_EP, LANES), lambda i: (i, 0)),
            pl.BlockSpec((BN_EP, LANES), lambda i: (i, 0)),
            pl.BlockSpec((BN_EP, H1), lambda i: (i, 0)),
            pl.BlockSpec((H1, H1), lambda i: (0, 0)),
            pl.BlockSpec((1, H1), lambda i: (0, 0)),
        ],
        out_specs=[
            pl.BlockSpec((BN_EP, H2 // 2), lambda i: (i, 0)),
            pl.BlockSpec((BN_EP, H2 // 2), lambda i: (i, 0)),
            pl.BlockSpec((BN_EP, H2), lambda i: (i, 0)),
            pl.BlockSpec((BN_EP, 1), lambda i: (i, 0)),
        ],
        out_shape=[
            jax.ShapeDtypeStruct((N, H2 // 2), jnp.float32),
            jax.ShapeDtypeStruct((N, H2 // 2), jnp.float32),
            jax.ShapeDtypeStruct((N, H2), jnp.float32),
            jax.ShapeDtypeStruct((N, 1), jnp.float32),
        ],
    )(accA, accB, deg0, deg1, xr, w2cat, b1r)


def _ep2_body(accA_ref, accB_ref, recip_ref, gr_ref, b2_ref, out_ref):
    summed = jnp.concatenate([accA_ref[...], accB_ref[...]], axis=1)
    h2 = summed * recip_ref[...] + b2_ref[...] + gr_ref[...]
    y = jnp.maximum(h2, 0.0)
    m = jnp.max(y, axis=1, keepdims=True)
    lse = jnp.log(jnp.sum(jnp.exp(y - m), axis=1, keepdims=True)) + m
    out_ref[...] = y - lse


def _ep2(accA, accB, recip, gr, b2r):
    return pl.pallas_call(
        _ep2_body,
        grid=(N // BN_EP,),
        in_specs=[
            pl.BlockSpec((BN_EP, H2 // 2), lambda i: (i, 0)),
            pl.BlockSpec((BN_EP, H2 // 2), lambda i: (i, 0)),
            pl.BlockSpec((BN_EP, 1), lambda i: (i, 0)),
            pl.BlockSpec((BN_EP, H2), lambda i: (i, 0)),
            pl.BlockSpec((1, H2), lambda i: (0, 0)),
        ],
        out_specs=pl.BlockSpec((BN_EP, H2), lambda i: (i, 0)),
        out_shape=jax.ShapeDtypeStruct((N, H2), jnp.float32),
    )(accA, accB, recip, gr, b2r)


def kernel(x, edge_index, Wl1, Wr1, b1, Wl2, Wr2, b2):
    extra = EPAD - E
    src = jnp.concatenate(
        [edge_index[0], jnp.zeros((extra,), jnp.int32)]).reshape(ECH, CH)
    dst = jnp.concatenate(
        [edge_index[1], jnp.full((extra,), N, jnp.int32)]).reshape(ECH, CH)
    w1cat = jnp.concatenate([Wl1, Wr1], axis=1)
    w2cat = jnp.concatenate([Wl2, Wr2], axis=1)
    b1r = b1.reshape(1, H1)
    b2r = b2.reshape(1, H2)

    hA, hB, xr = _mm1(x, w1cat)
    deg0, deg1 = _deg_kernel(dst)
    accA, accB = _agg1(hA, hB, src, dst)
    gA, gB, gr, recip = _ep1(accA, accB, deg0, deg1, xr, w2cat, b1r)
    acc2A, acc2B = _agg2(gA, gB, src, dst)
    return _ep2(acc2A, acc2B, recip, gr, b2r)


# speedup vs baseline: 8.1420x; 1.3081x over previous
import functools

import jax
import jax.numpy as jnp
from jax import lax
from jax.experimental import pallas as pl
from jax.experimental.pallas import tpu as pltpu
from jax.experimental.pallas import tpu_sc as plsc

N = 50000
E = 800000
D_IN = 1433
H1 = 64
H2 = 32

NC = 2
NS = 16
LANES = 16

CH = 128
EPAD = 802816
ECH = EPAD // CH
KB1 = 4
KB2 = 8
KD = 4
NBATCH_D = ECH // NC // NS // KD
NPAD = 50176
ROWS_PER_TILE = NPAD // NS
ZR = 392

_mesh = plsc.VectorSubcoreMesh(
    core_axis_name="c", subcore_axis_name="s", num_cores=NC, num_subcores=NS
)


def _zero_stage(stage_ref, width):
    nseg = width // LANES
    nrows = stage_ref.shape[0]

    def body(i, _):
        for j in range(nseg):
            stage_ref[i, pl.ds(j * LANES, LANES)] = jnp.zeros((LANES,), jnp.float32)
        return 0
    lax.fori_loop(0, nrows, body, 0, unroll=4)


def _deg_body(dst_hbm, out0_hbm, out1_hbm, deg_sh, idx_d2, ones_v, stage_v, sem):
    c = lax.axis_index("c")
    s = lax.axis_index("s")

    def fill_ones(i, _):
        ones_v[i, :] = jnp.ones((LANES,), jnp.float32)
        return 0
    lax.fori_loop(0, CH, fill_ones, 0, unroll=4)
    _zero_stage(stage_v, LANES)

    r0 = s * ROWS_PER_TILE

    def zero_acc(j, _):
        pltpu.sync_copy(stage_v, deg_sh.at[pl.ds(r0 + j * ZR, ZR)])
        return 0
    lax.fori_loop(0, ROWS_PER_TILE // ZR, zero_acc, 0)
    plsc.subcore_barrier()

    base = (c * NS + s) * (ECH // NC // NS)

    def edge_loop(b, _):
        r = base + b * KD
        pltpu.async_copy(dst_hbm.at[pl.ds(r, KD)], idx_d2, sem).wait()
        descs = [
            pltpu.async_copy(ones_v, deg_sh.at[idx_d2.at[j]], sem, add=True)
            for j in range(KD)
        ]
        for d in descs:
            d.wait()
        return 0
    lax.fori_loop(0, NBATCH_D, edge_loop, 0)
    plsc.subcore_barrier()

    def out_loop(j, _):
        rr = r0 + j * ZR
        pltpu.sync_copy(deg_sh.at[pl.ds(rr, ZR)], stage_v)

        @pl.when(c == 0)
        def _():
            pltpu.sync_copy(stage_v, out0_hbm.at[pl.ds(rr, ZR)])

        @pl.when(c == 1)
        def _():
            pltpu.sync_copy(stage_v, out1_hbm.at[pl.ds(rr, ZR)])
        return 0
    lax.fori_loop(0, ROWS_PER_TILE // ZR, out_loop, 0)


_deg_kernel = functools.partial(
    pl.kernel,
    out_type=[
        jax.ShapeDtypeStruct((NPAD, LANES), jnp.float32),
        jax.ShapeDtypeStruct((NPAD, LANES), jnp.float32),
    ],
    mesh=_mesh,
    compiler_params=pltpu.CompilerParams(use_tc_tiling_on_sc=False),
    scratch_types=[
        pltpu.VMEM_SHARED((NPAD, LANES), jnp.float32),
        pltpu.VMEM((KD, CH), jnp.int32),
        pltpu.VMEM((CH, LANES), jnp.float32),
        pltpu.VMEM((ZR, LANES), jnp.float32),
        pltpu.SemaphoreType.DMA,
    ],
)(_deg_body)


def _agg_body(dh, kb, tA_hbm, tB_hbm, src_hbm, dst_hbm, outA_hbm, outB_hbm,
              acc_sh, idx_s2, idx_d2, rows2, sem):
    c = lax.axis_index("c")
    s = lax.axis_index("s")

    _zero_stage(rows2, dh)
    r0 = s * ROWS_PER_TILE

    def zero_acc(j, _):
        pltpu.sync_copy(rows2.at[pl.ds(0, ZR)], acc_sh.at[pl.ds(r0 + j * ZR, ZR)])
        return 0
    lax.fori_loop(0, ROWS_PER_TILE // ZR, zero_acc, 0)
    plsc.subcore_barrier()

    base = s * (ECH // NS)
    nbatch = ECH // NS // kb

    def edge_loop(b, _):
        r = base + b * kb
        di = pltpu.async_copy(src_hbm.at[pl.ds(r, kb)], idx_s2, sem)
        dj = pltpu.async_copy(dst_hbm.at[pl.ds(r, kb)], idx_d2, sem)
        di.wait()
        dj.wait()

        @pl.when(c == 0)
        def _():
            descs = [
                pltpu.async_copy(
                    tA_hbm.at[idx_s2.at[j]],
                    rows2.at[pl.ds(j * CH, CH)], sem)
                for j in range(kb)
            ]
            for d in descs:
                d.wait()

        @pl.when(c == 1)
        def _():
            descs = [
                pltpu.async_copy(
                    tB_hbm.at[idx_s2.at[j]],
                    rows2.at[pl.ds(j * CH, CH)], sem)
                for j in range(kb)
            ]
            for d in descs:
                d.wait()

        descs = [
            pltpu.async_copy(
                rows2.at[pl.ds(j * CH, CH)],
                acc_sh.at[idx_d2.at[j]], sem, add=True)
            for j in range(kb)
        ]
        for d in descs:
            d.wait()
        return 0
    lax.fori_loop(0, nbatch, edge_loop, 0)
    plsc.subcore_barrier()

    def out_loop(j, _):
        rr = r0 + j * ZR
        pltpu.sync_copy(acc_sh.at[pl.ds(rr, ZR)], rows2.at[pl.ds(0, ZR)])

        @pl.when(c == 0)
        def _():
            pltpu.sync_copy(rows2.at[pl.ds(0, ZR)], outA_hbm.at[pl.ds(rr, ZR)])

        @pl.when(c == 1)
        def _():
            pltpu.sync_copy(rows2.at[pl.ds(0, ZR)], outB_hbm.at[pl.ds(rr, ZR)])
        return 0
    lax.fori_loop(0, ROWS_PER_TILE // ZR, out_loop, 0)


def _make_agg(dh, kb):
    return functools.partial(
        pl.kernel,
        out_type=[
            jax.ShapeDtypeStruct((NPAD, dh), jnp.float32),
            jax.ShapeDtypeStruct((NPAD, dh), jnp.float32),
        ],
        mesh=_mesh,
        compiler_params=pltpu.CompilerParams(use_tc_tiling_on_sc=False),
        scratch_types=[
            pltpu.VMEM_SHARED((NPAD, dh), jnp.float32),
            pltpu.VMEM((kb, CH), jnp.int32),
            pltpu.VMEM((kb, CH), jnp.int32),
            pltpu.VMEM((kb * CH, dh), jnp.float32),
            pltpu.SemaphoreType.DMA,
        ],
    )(functools.partial(_agg_body, dh, kb))


_agg1 = _make_agg(H1 // 2, KB1)
_agg2 = _make_agg(H2 // 2, KB2)


BN_MM = 512
BN_EP = 1000


def _mm1_body(xt_ref, w_ref, hA_ref, hB_ref, xr_ref):
    g = lax.dot_general(
        xt_ref[...], w_ref[...], (((0,), (0,)), ((), ())),
        preferred_element_type=jnp.float32)
    hA_ref[...] = g[:, 0:H1 // 2]
    hB_ref[...] = g[:, H1 // 2:H1]
    xr_ref[...] = g[:, H1:]


def _mm1(xt, w1cat):
    return pl.pallas_call(
        _mm1_body,
        grid=(pl.cdiv(N, BN_MM),),
        in_specs=[
            pl.BlockSpec((D_IN, BN_MM), lambda i: (0, i)),
            pl.BlockSpec((D_IN, 2 * H1), lambda i: (0, 0)),
        ],
        out_specs=[
            pl.BlockSpec((BN_MM, H1 // 2), lambda i: (i, 0)),
            pl.BlockSpec((BN_MM, H1 // 2), lambda i: (i, 0)),
            pl.BlockSpec((BN_MM, H1), lambda i: (i, 0)),
        ],
        out_shape=[
            jax.ShapeDtypeStruct((N, H1 // 2), jnp.float32),
            jax.ShapeDtypeStruct((N, H1 // 2), jnp.float32),
            jax.ShapeDtypeStruct((N, H1), jnp.float32),
        ],
    )(xt, w1cat)


def _ep1_body(accA_ref, accB_ref, deg0_ref, deg1_ref, xr_ref, w2_ref, b1_ref,
              gA_ref, gB_ref, gr_ref, recip_ref):
    deg = deg0_ref[:, 0:1] + deg1_ref[:, 0:1]
    recip = 1.0 / jnp.maximum(deg, 1.0)
    summed = jnp.concatenate([accA_ref[...], accB_ref[...]], axis=1)
    h1 = summed * recip + b1_ref[...] + xr_ref[...]
    g = jnp.dot(h1, w2_ref[...], preferred_element_type=jnp.float32)
    gA_ref[...] = g[:, 0:H2 // 2]
    gB_ref[...] = g[:, H2 // 2:H2]
    gr_ref[...] = g[:, H2:]
    recip_ref[...] = recip


def _ep1(accA, accB, deg0, deg1, xr, w2cat, b1r):
    return pl.pallas_call(
        _ep1_body,
        grid=(N // BN_EP,),
        in_specs=[
            pl.BlockSpec((BN_EP, H1 // 2), lambda i: (i, 0)),
            pl.BlockSpec((BN_EP, H1 // 2), lambda i: (i, 0)),
            pl.BlockSpec((BN_EP, LANES), lambda i: (i, 0)),
            pl.BlockSpec((BN_EP, LANES), lambda i: (i, 0)),
            pl.BlockSpec((BN_EP, H1), lambda i: (i, 0)),
            pl.BlockSpec((H1, H1), lambda i: (0, 0)),
            pl.BlockSpec((1, H1), lambda i: (0, 0)),
        ],
        out_specs=[
            pl.BlockSpec((BN_EP, H2 // 2), lambda i: (i, 0)),
            pl.BlockSpec((BN_EP, H2 // 2), lambda i: (i, 0)),
            pl.BlockSpec((BN_EP, H2), lambda i: (i, 0)),
            pl.BlockSpec((BN_EP, 1), lambda i: (i, 0)),
        ],
        out_shape=[
            jax.ShapeDtypeStruct((N, H2 // 2), jnp.float32),
            jax.ShapeDtypeStruct((N, H2 // 2), jnp.float32),
            jax.ShapeDtypeStruct((N, H2), jnp.float32),
            jax.ShapeDtypeStruct((N, 1), jnp.float32),
        ],
    )(accA, accB, deg0, deg1, xr, w2cat, b1r)


def _ep2_body(accA_ref, accB_ref, recip_ref, gr_ref, b2_ref, out_ref):
    summed = jnp.concatenate([accA_ref[...], accB_ref[...]], axis=1)
    h2 = summed * recip_ref[...] + b2_ref[...] + gr_ref[...]
    y = jnp.maximum(h2, 0.0)
    m = jnp.max(y, axis=1, keepdims=True)
    lse = jnp.log(jnp.sum(jnp.exp(y - m), axis=1, keepdims=True)) + m
    out_ref[...] = y - lse


def _ep2(accA, accB, recip, gr, b2r):
    return pl.pallas_call(
        _ep2_body,
        grid=(N // BN_EP,),
        in_specs=[
            pl.BlockSpec((BN_EP, H2 // 2), lambda i: (i, 0)),
            pl.BlockSpec((BN_EP, H2 // 2), lambda i: (i, 0)),
            pl.BlockSpec((BN_EP, 1), lambda i: (i, 0)),
            pl.BlockSpec((BN_EP, H2), lambda i: (i, 0)),
            pl.BlockSpec((1, H2), lambda i: (0, 0)),
        ],
        out_specs=pl.BlockSpec((BN_EP, H2), lambda i: (i, 0)),
        out_shape=jax.ShapeDtypeStruct((N, H2), jnp.float32),
    )(accA, accB, recip, gr, b2r)


def kernel(x, edge_index, Wl1, Wr1, b1, Wl2, Wr2, b2):
    extra = EPAD - E
    src = jnp.concatenate(
        [edge_index[0], jnp.zeros((extra,), jnp.int32)]).reshape(ECH, CH)
    dst = jnp.concatenate(
        [edge_index[1], jnp.full((extra,), N, jnp.int32)]).reshape(ECH, CH)
    w1cat = jnp.concatenate([Wl1, Wr1], axis=1)
    w2cat = jnp.concatenate([Wl2, Wr2], axis=1)
    b1r = b1.reshape(1, H1)
    b2r = b2.reshape(1, H2)

    hA, hB, xr = _mm1(x.T, w1cat)
    deg0, deg1 = _deg_kernel(dst)
    accA, accB = _agg1(hA, hB, src, dst)
    gA, gB, gr, recip = _ep1(accA, accB, deg0, deg1, xr, w2cat, b1r)
    acc2A, acc2B = _agg2(gA, gB, src, dst)
    return _ep2(acc2A, acc2B, recip, gr, b2r)


# speedup vs baseline: 9.3879x; 1.1530x over previous
import functools

import jax
import jax.numpy as jnp
from jax import lax
from jax.experimental import pallas as pl
from jax.experimental.pallas import tpu as pltpu
from jax.experimental.pallas import tpu_sc as plsc

N = 50000
E = 800000
D_IN = 1433
H1 = 64
H2 = 32

NC = 2
NS = 16
LANES = 16

CH = 128
EPAD = 802816
ECH = EPAD // CH
KB1 = 4
KB2 = 8
KD = 4
NBATCH_D = ECH // NC // NS // KD
NPAD = 50176
ROWS_PER_TILE = NPAD // NS
ZR = 392

_mesh = plsc.VectorSubcoreMesh(
    core_axis_name="c", subcore_axis_name="s", num_cores=NC, num_subcores=NS
)


def _zero_stage(stage_ref, width):
    nseg = width // LANES
    nrows = stage_ref.shape[0]

    def body(i, _):
        for j in range(nseg):
            stage_ref[i, pl.ds(j * LANES, LANES)] = jnp.zeros((LANES,), jnp.float32)
        return 0
    lax.fori_loop(0, nrows, body, 0, unroll=4)


def _deg_body(dst_hbm, out0_hbm, out1_hbm, deg_sh, idx_d2, ones_v, stage_v, sem):
    c = lax.axis_index("c")
    s = lax.axis_index("s")

    def fill_ones(i, _):
        ones_v[i, :] = jnp.ones((LANES,), jnp.float32)
        return 0
    lax.fori_loop(0, CH, fill_ones, 0, unroll=4)
    _zero_stage(stage_v, LANES)

    r0 = s * ROWS_PER_TILE

    def zero_acc(j, _):
        pltpu.sync_copy(stage_v, deg_sh.at[pl.ds(r0 + j * ZR, ZR)])
        return 0
    lax.fori_loop(0, ROWS_PER_TILE // ZR, zero_acc, 0)
    plsc.subcore_barrier()

    base = (c * NS + s) * (ECH // NC // NS)

    def edge_loop(b, _):
        r = base + b * KD
        pltpu.async_copy(dst_hbm.at[pl.ds(r, KD)], idx_d2, sem).wait()
        descs = [
            pltpu.async_copy(ones_v, deg_sh.at[idx_d2.at[j]], sem, add=True)
            for j in range(KD)
        ]
        for d in descs:
            d.wait()
        return 0
    lax.fori_loop(0, NBATCH_D, edge_loop, 0)
    plsc.subcore_barrier()

    def out_loop(j, _):
        rr = r0 + j * ZR
        pltpu.sync_copy(deg_sh.at[pl.ds(rr, ZR)], stage_v)

        @pl.when(c == 0)
        def _():
            pltpu.sync_copy(stage_v, out0_hbm.at[pl.ds(rr, ZR)])

        @pl.when(c == 1)
        def _():
            pltpu.sync_copy(stage_v, out1_hbm.at[pl.ds(rr, ZR)])
        return 0
    lax.fori_loop(0, ROWS_PER_TILE // ZR, out_loop, 0)


_deg_kernel = functools.partial(
    pl.kernel,
    out_type=[
        jax.ShapeDtypeStruct((NPAD, LANES), jnp.float32),
        jax.ShapeDtypeStruct((NPAD, LANES), jnp.float32),
    ],
    mesh=_mesh,
    compiler_params=pltpu.CompilerParams(use_tc_tiling_on_sc=False),
    scratch_types=[
        pltpu.VMEM_SHARED((NPAD, LANES), jnp.float32),
        pltpu.VMEM((KD, CH), jnp.int32),
        pltpu.VMEM((CH, LANES), jnp.float32),
        pltpu.VMEM((ZR, LANES), jnp.float32),
        pltpu.SemaphoreType.DMA,
    ],
)(_deg_body)


def _agg_body(dh, kb, tA_hbm, tB_hbm, src_hbm, dst_hbm, outA_hbm, outB_hbm,
              acc_sh, idx_s2, idx_d2, rows2, sem_i, sem_s, *sem_g):
    c = lax.axis_index("c")
    s = lax.axis_index("s")

    _zero_stage(rows2, dh)
    r0 = s * ROWS_PER_TILE

    def zero_acc(j, _):
        pltpu.sync_copy(rows2.at[pl.ds(0, ZR)], acc_sh.at[pl.ds(r0 + j * ZR, ZR)])
        return 0
    lax.fori_loop(0, ROWS_PER_TILE // ZR, zero_acc, 0)
    plsc.subcore_barrier()

    base = s * (ECH // NS)
    nbatch = ECH // NS // kb

    pltpu.async_copy(src_hbm.at[pl.ds(base, kb)], idx_s2.at[0], sem_i)
    pltpu.async_copy(dst_hbm.at[pl.ds(base, kb)], idx_d2.at[0], sem_i)

    def pipeline(table, p, rnext):
        gd = [
            pltpu.async_copy(
                table.at[idx_s2.at[p, j]],
                rows2.at[pl.ds(j * CH, CH)], sem_g[j])
            for j in range(kb)
        ]
        pltpu.async_copy(src_hbm.at[pl.ds(rnext, kb)], idx_s2.at[1 - p], sem_i)
        pltpu.async_copy(dst_hbm.at[pl.ds(rnext, kb)], idx_d2.at[1 - p], sem_i)
        sc = []
        for j in range(kb):
            gd[j].wait()
            sc.append(pltpu.async_copy(
                rows2.at[pl.ds(j * CH, CH)],
                acc_sh.at[idx_d2.at[p, j]], sem_s, add=True))
        for d in sc:
            d.wait()

    def edge_loop(b, _):
        p = b % 2
        pltpu.make_async_copy(
            src_hbm.at[pl.ds(base, kb)], idx_s2.at[p], sem_i).wait()
        pltpu.make_async_copy(
            dst_hbm.at[pl.ds(base, kb)], idx_d2.at[p], sem_i).wait()
        rnext = base + ((b + 1) % nbatch) * kb

        @pl.when(c == 0)
        def _():
            pipeline(tA_hbm, p, rnext)

        @pl.when(c == 1)
        def _():
            pipeline(tB_hbm, p, rnext)
        return 0
    lax.fori_loop(0, nbatch, edge_loop, 0)
    pltpu.make_async_copy(
        src_hbm.at[pl.ds(base, kb)], idx_s2.at[nbatch % 2], sem_i).wait()
    pltpu.make_async_copy(
        dst_hbm.at[pl.ds(base, kb)], idx_d2.at[nbatch % 2], sem_i).wait()
    plsc.subcore_barrier()

    def out_loop(j, _):
        rr = r0 + j * ZR
        pltpu.sync_copy(acc_sh.at[pl.ds(rr, ZR)], rows2.at[pl.ds(0, ZR)])

        @pl.when(c == 0)
        def _():
            pltpu.sync_copy(rows2.at[pl.ds(0, ZR)], outA_hbm.at[pl.ds(rr, ZR)])

        @pl.when(c == 1)
        def _():
            pltpu.sync_copy(rows2.at[pl.ds(0, ZR)], outB_hbm.at[pl.ds(rr, ZR)])
        return 0
    lax.fori_loop(0, ROWS_PER_TILE // ZR, out_loop, 0)


def _make_agg(dh, kb):
    return functools.partial(
        pl.kernel,
        out_type=[
            jax.ShapeDtypeStruct((NPAD, dh), jnp.float32),
            jax.ShapeDtypeStruct((NPAD, dh), jnp.float32),
        ],
        mesh=_mesh,
        compiler_params=pltpu.CompilerParams(use_tc_tiling_on_sc=False),
        scratch_types=[
            pltpu.VMEM_SHARED((NPAD, dh), jnp.float32),
            pltpu.VMEM((2, kb, CH), jnp.int32),
            pltpu.VMEM((2, kb, CH), jnp.int32),
            pltpu.VMEM((kb * CH, dh), jnp.float32),
            pltpu.SemaphoreType.DMA,
            pltpu.SemaphoreType.DMA,
        ] + [pltpu.SemaphoreType.DMA] * kb,
    )(functools.partial(_agg_body, dh, kb))


_agg1 = _make_agg(H1 // 2, KB1)
_agg2 = _make_agg(H2 // 2, KB2)


BN_MM = 512
BN_EP = 1000


def _mm1_body(xt_ref, w_ref, hA_ref, hB_ref, xr_ref):
    g = lax.dot_general(
        xt_ref[...], w_ref[...], (((0,), (0,)), ((), ())),
        preferred_element_type=jnp.float32)
    hA_ref[...] = g[:, 0:H1 // 2]
    hB_ref[...] = g[:, H1 // 2:H1]
    xr_ref[...] = g[:, H1:]


def _mm1(xt, w1cat):
    return pl.pallas_call(
        _mm1_body,
        grid=(pl.cdiv(N, BN_MM),),
        in_specs=[
            pl.BlockSpec((D_IN, BN_MM), lambda i: (0, i)),
            pl.BlockSpec((D_IN, 2 * H1), lambda i: (0, 0)),
        ],
        out_specs=[
            pl.BlockSpec((BN_MM, H1 // 2), lambda i: (i, 0)),
            pl.BlockSpec((BN_MM, H1 // 2), lambda i: (i, 0)),
            pl.BlockSpec((BN_MM, H1), lambda i: (i, 0)),
        ],
        out_shape=[
            jax.ShapeDtypeStruct((N, H1 // 2), jnp.float32),
            jax.ShapeDtypeStruct((N, H1 // 2), jnp.float32),
            jax.ShapeDtypeStruct((N, H1), jnp.float32),
        ],
    )(xt, w1cat)


def _ep1_body(accA_ref, accB_ref, deg0_ref, deg1_ref, xr_ref, w2_ref, b1_ref,
              gA_ref, gB_ref, gr_ref, recip_ref):
    deg = deg0_ref[:, 0:1] + deg1_ref[:, 0:1]
    recip = 1.0 / jnp.maximum(deg, 1.0)
    summed = jnp.concatenate([accA_ref[...], accB_ref[...]], axis=1)
    h1 = summed * recip + b1_ref[...] + xr_ref[...]
    g = jnp.dot(h1, w2_ref[...], preferred_element_type=jnp.float32)
    gA_ref[...] = g[:, 0:H2 // 2]
    gB_ref[...] = g[:, H2 // 2:H2]
    gr_ref[...] = g[:, H2:]
    recip_ref[...] = recip


def _ep1(accA, accB, deg0, deg1, xr, w2cat, b1r):
    return pl.pallas_call(
        _ep1_body,
        grid=(N // BN_EP,),
        in_specs=[
            pl.BlockSpec((BN_EP, H1 // 2), lambda i: (i, 0)),
            pl.BlockSpec((BN_EP, H1 // 2), lambda i: (i, 0)),
            pl.BlockSpec((BN_EP, LANES), lambda i: (i, 0)),
            pl.BlockSpec((BN_EP, LANES), lambda i: (i, 0)),
            pl.BlockSpec((BN_EP, H1), lambda i: (i, 0)),
            pl.BlockSpec((H1, H1), lambda i: (0, 0)),
            pl.BlockSpec((1, H1), lambda i: (0, 0)),
        ],
        out_specs=[
            pl.BlockSpec((BN_EP, H2 // 2), lambda i: (i, 0)),
            pl.BlockSpec((BN_EP, H2 // 2), lambda i: (i, 0)),
            pl.BlockSpec((BN_EP, H2), lambda i: (i, 0)),
            pl.BlockSpec((BN_EP, 1), lambda i: (i, 0)),
        ],
        out_shape=[
            jax.ShapeDtypeStruct((N, H2 // 2), jnp.float32),
            jax.ShapeDtypeStruct((N, H2 // 2), jnp.float32),
            jax.ShapeDtypeStruct((N, H2), jnp.float32),
            jax.ShapeDtypeStruct((N, 1), jnp.float32),
        ],
    )(accA, accB, deg0, deg1, xr, w2cat, b1r)


def _ep2_body(accA_ref, accB_ref, recip_ref, gr_ref, b2_ref, out_ref):
    summed = jnp.concatenate([accA_ref[...], accB_ref[...]], axis=1)
    h2 = summed * recip_ref[...] + b2_ref[...] + gr_ref[...]
    y = jnp.maximum(h2, 0.0)
    m = jnp.max(y, axis=1, keepdims=True)
    lse = jnp.log(jnp.sum(jnp.exp(y - m), axis=1, keepdims=True)) + m
    out_ref[...] = y - lse


def _ep2(accA, accB, recip, gr, b2r):
    return pl.pallas_call(
        _ep2_body,
        grid=(N // BN_EP,),
        in_specs=[
            pl.BlockSpec((BN_EP, H2 // 2), lambda i: (i, 0)),
            pl.BlockSpec((BN_EP, H2 // 2), lambda i: (i, 0)),
            pl.BlockSpec((BN_EP, 1), lambda i: (i, 0)),
            pl.BlockSpec((BN_EP, H2), lambda i: (i, 0)),
            pl.BlockSpec((1, H2), lambda i: (0, 0)),
        ],
        out_specs=pl.BlockSpec((BN_EP, H2), lambda i: (i, 0)),
        out_shape=jax.ShapeDtypeStruct((N, H2), jnp.float32),
    )(accA, accB, recip, gr, b2r)


def kernel(x, edge_index, Wl1, Wr1, b1, Wl2, Wr2, b2):
    extra = EPAD - E
    src = jnp.concatenate(
        [edge_index[0], jnp.zeros((extra,), jnp.int32)]).reshape(ECH, CH)
    dst = jnp.concatenate(
        [edge_index[1], jnp.full((extra,), N, jnp.int32)]).reshape(ECH, CH)
    w1cat = jnp.concatenate([Wl1, Wr1], axis=1)
    w2cat = jnp.concatenate([Wl2, Wr2], axis=1)
    b1r = b1.reshape(1, H1)
    b2r = b2.reshape(1, H2)

    hA, hB, xr = _mm1(x.T, w1cat)
    deg0, deg1 = _deg_kernel(dst)
    accA, accB = _agg1(hA, hB, src, dst)
    gA, gB, gr, recip = _ep1(accA, accB, deg0, deg1, xr, w2cat, b1r)
    acc2A, acc2B = _agg2(gA, gB, src, dst)
    return _ep2(acc2A, acc2B, recip, gr, b2r)


# speedup vs baseline: 10.0473x; 1.0702x over previous
import functools

import jax
import jax.numpy as jnp
from jax import lax
from jax.experimental import pallas as pl
from jax.experimental.pallas import tpu as pltpu
from jax.experimental.pallas import tpu_sc as plsc

N = 50000
E = 800000
D_IN = 1433
H1 = 64
H2 = 32

NC = 2
NS = 16
LANES = 16

CH = 128
EPAD = 802816
ECH = EPAD // CH
KB1 = 4
KB2 = 8
KD = 4
NBATCH_D = ECH // NC // NS // KD
NPAD = 50176
ROWS_PER_TILE = NPAD // NS
ZR = 392

_mesh = plsc.VectorSubcoreMesh(
    core_axis_name="c", subcore_axis_name="s", num_cores=NC, num_subcores=NS
)


def _zero_stage(stage_ref, width):
    nseg = width // LANES
    nrows = stage_ref.shape[0]

    def body(i, _):
        for j in range(nseg):
            stage_ref[i, pl.ds(j * LANES, LANES)] = jnp.zeros((LANES,), jnp.float32)
        return 0
    lax.fori_loop(0, nrows, body, 0, unroll=4)


def _deg_body(dst_hbm, out0_hbm, out1_hbm, deg_sh, idx_d2, ones_v, stage_v, sem):
    c = lax.axis_index("c")
    s = lax.axis_index("s")

    def fill_ones(i, _):
        ones_v[i, :] = jnp.ones((LANES,), jnp.float32)
        return 0
    lax.fori_loop(0, CH, fill_ones, 0, unroll=4)
    _zero_stage(stage_v, LANES)

    r0 = s * ROWS_PER_TILE

    def zero_acc(j, _):
        pltpu.sync_copy(stage_v, deg_sh.at[pl.ds(r0 + j * ZR, ZR)])
        return 0
    lax.fori_loop(0, ROWS_PER_TILE // ZR, zero_acc, 0)
    plsc.subcore_barrier()

    base = (c * NS + s) * (ECH // NC // NS)

    def edge_loop(b, _):
        r = base + b * KD
        pltpu.async_copy(dst_hbm.at[pl.ds(r, KD)], idx_d2, sem).wait()
        descs = [
            pltpu.async_copy(ones_v, deg_sh.at[idx_d2.at[j]], sem, add=True)
            for j in range(KD)
        ]
        for d in descs:
            d.wait()
        return 0
    lax.fori_loop(0, NBATCH_D, edge_loop, 0)
    plsc.subcore_barrier()

    def out_loop(j, _):
        rr = r0 + j * ZR
        pltpu.sync_copy(deg_sh.at[pl.ds(rr, ZR)], stage_v)

        @pl.when(c == 0)
        def _():
            pltpu.sync_copy(stage_v, out0_hbm.at[pl.ds(rr, ZR)])

        @pl.when(c == 1)
        def _():
            pltpu.sync_copy(stage_v, out1_hbm.at[pl.ds(rr, ZR)])
        return 0
    lax.fori_loop(0, ROWS_PER_TILE // ZR, out_loop, 0)


_deg_kernel = functools.partial(
    pl.kernel,
    out_type=[
        jax.ShapeDtypeStruct((NPAD, LANES), jnp.float32),
        jax.ShapeDtypeStruct((NPAD, LANES), jnp.float32),
    ],
    mesh=_mesh,
    compiler_params=pltpu.CompilerParams(use_tc_tiling_on_sc=False),
    scratch_types=[
        pltpu.VMEM_SHARED((NPAD, LANES), jnp.float32),
        pltpu.VMEM((KD, CH), jnp.int32),
        pltpu.VMEM((CH, LANES), jnp.float32),
        pltpu.VMEM((ZR, LANES), jnp.float32),
        pltpu.SemaphoreType.DMA,
    ],
)(_deg_body)


def _agg_body(dh, kb, tA_hbm, tB_hbm, src_hbm, dst_hbm, outA_hbm, outB_hbm,
              acc_sh, idx_s2, idx_d2, rows2, sem_i, sem_s, *sem_g):
    c = lax.axis_index("c")
    s = lax.axis_index("s")

    _zero_stage(rows2, dh)
    r0 = s * ROWS_PER_TILE

    def zero_acc(j, _):
        pltpu.sync_copy(rows2.at[pl.ds(0, ZR)], acc_sh.at[pl.ds(r0 + j * ZR, ZR)])
        return 0
    lax.fori_loop(0, ROWS_PER_TILE // ZR, zero_acc, 0)
    plsc.subcore_barrier()

    base = s * (ECH // NS)
    nbatch = ECH // NS // kb

    pltpu.async_copy(src_hbm.at[pl.ds(base, kb)], idx_s2.at[0], sem_i)
    pltpu.async_copy(dst_hbm.at[pl.ds(base, kb)], idx_d2.at[0], sem_i)

    def pipeline(table, p, rnext):
        gd = [
            pltpu.async_copy(
                table.at[idx_s2.at[p, j]],
                rows2.at[pl.ds(j * CH, CH)], sem_g[j])
            for j in range(kb)
        ]
        pltpu.async_copy(src_hbm.at[pl.ds(rnext, kb)], idx_s2.at[1 - p], sem_i)
        pltpu.async_copy(dst_hbm.at[pl.ds(rnext, kb)], idx_d2.at[1 - p], sem_i)
        sc = []
        for j in range(kb):
            gd[j].wait()
            sc.append(pltpu.async_copy(
                rows2.at[pl.ds(j * CH, CH)],
                acc_sh.at[idx_d2.at[p, j]], sem_s, add=True))
        for d in sc:
            d.wait()

    def edge_loop(b, _):
        p = b % 2
        pltpu.make_async_copy(
            src_hbm.at[pl.ds(base, kb)], idx_s2.at[p], sem_i).wait()
        pltpu.make_async_copy(
            dst_hbm.at[pl.ds(base, kb)], idx_d2.at[p], sem_i).wait()
        rnext = base + ((b + 1) % nbatch) * kb

        @pl.when(c == 0)
        def _():
            pipeline(tA_hbm, p, rnext)

        @pl.when(c == 1)
        def _():
            pipeline(tB_hbm, p, rnext)
        return 0
    lax.fori_loop(0, nbatch, edge_loop, 0)
    pltpu.make_async_copy(
        src_hbm.at[pl.ds(base, kb)], idx_s2.at[nbatch % 2], sem_i).wait()
    pltpu.make_async_copy(
        dst_hbm.at[pl.ds(base, kb)], idx_d2.at[nbatch % 2], sem_i).wait()
    plsc.subcore_barrier()

    def out_loop(j, _):
        rr = r0 + j * ZR
        pltpu.sync_copy(acc_sh.at[pl.ds(rr, ZR)], rows2.at[pl.ds(0, ZR)])

        @pl.when(c == 0)
        def _():
            pltpu.sync_copy(rows2.at[pl.ds(0, ZR)], outA_hbm.at[pl.ds(rr, ZR)])

        @pl.when(c == 1)
        def _():
            pltpu.sync_copy(rows2.at[pl.ds(0, ZR)], outB_hbm.at[pl.ds(rr, ZR)])
        return 0
    lax.fori_loop(0, ROWS_PER_TILE // ZR, out_loop, 0)


def _make_agg(dh, kb):
    return functools.partial(
        pl.kernel,
        out_type=[
            jax.ShapeDtypeStruct((NPAD, dh), jnp.float32),
            jax.ShapeDtypeStruct((NPAD, dh), jnp.float32),
        ],
        mesh=_mesh,
        compiler_params=pltpu.CompilerParams(use_tc_tiling_on_sc=False),
        scratch_types=[
            pltpu.VMEM_SHARED((NPAD, dh), jnp.float32),
            pltpu.VMEM((2, kb, CH), jnp.int32),
            pltpu.VMEM((2, kb, CH), jnp.int32),
            pltpu.VMEM((kb * CH, dh), jnp.float32),
            pltpu.SemaphoreType.DMA,
            pltpu.SemaphoreType.DMA,
        ] + [pltpu.SemaphoreType.DMA] * kb,
    )(functools.partial(_agg_body, dh, kb))


_agg1 = _make_agg(H1 // 2, KB1)
_agg2 = _make_agg(H2 // 2, KB2)


BN_MM = 1024
BN_EP = 2000


def _mm1_body(xt_ref, w_ref, hA_ref, hB_ref, xr_ref):
    g = lax.dot_general(
        xt_ref[...], w_ref[...], (((0,), (0,)), ((), ())),
        preferred_element_type=jnp.float32)
    hA_ref[...] = g[:, 0:H1 // 2]
    hB_ref[...] = g[:, H1 // 2:H1]
    xr_ref[...] = g[:, H1:]


def _mm1(xt, w1cat):
    return pl.pallas_call(
        _mm1_body,
        grid=(pl.cdiv(N, BN_MM),),
        in_specs=[
            pl.BlockSpec((D_IN, BN_MM), lambda i: (0, i)),
            pl.BlockSpec((D_IN, 2 * H1), lambda i: (0, 0)),
        ],
        out_specs=[
            pl.BlockSpec((BN_MM, H1 // 2), lambda i: (i, 0)),
            pl.BlockSpec((BN_MM, H1 // 2), lambda i: (i, 0)),
            pl.BlockSpec((BN_MM, H1), lambda i: (i, 0)),
        ],
        out_shape=[
            jax.ShapeDtypeStruct((N, H1 // 2), jnp.float32),
            jax.ShapeDtypeStruct((N, H1 // 2), jnp.float32),
            jax.ShapeDtypeStruct((N, H1), jnp.float32),
        ],
    )(xt, w1cat)


def _ep1_body(accA_ref, accB_ref, deg0_ref, deg1_ref, xr_ref, w2_ref, b1_ref,
              gA_ref, gB_ref, gr_ref, recip_ref):
    deg = deg0_ref[:, 0:1] + deg1_ref[:, 0:1]
    recip = 1.0 / jnp.maximum(deg, 1.0)
    summed = jnp.concatenate([accA_ref[...], accB_ref[...]], axis=1)
    h1 = summed * recip + b1_ref[...] + xr_ref[...]
    g = jnp.dot(h1, w2_ref[...], preferred_element_type=jnp.float32)
    gA_ref[...] = g[:, 0:H2 // 2]
    gB_ref[...] = g[:, H2 // 2:H2]
    gr_ref[...] = g[:, H2:]
    recip_ref[...] = recip


def _ep1(accA, accB, deg0, deg1, xr, w2cat, b1r):
    return pl.pallas_call(
        _ep1_body,
        grid=(N // BN_EP,),
        in_specs=[
            pl.BlockSpec((BN_EP, H1 // 2), lambda i: (i, 0)),
            pl.BlockSpec((BN_EP, H1 // 2), lambda i: (i, 0)),
            pl.BlockSpec((BN_EP, LANES), lambda i: (i, 0)),
            pl.BlockSpec((BN_EP, LANES), lambda i: (i, 0)),
            pl.BlockSpec((BN_EP, H1), lambda i: (i, 0)),
            pl.BlockSpec((H1, H1), lambda i: (0, 0)),
            pl.BlockSpec((1, H1), lambda i: (0, 0)),
        ],
        out_specs=[
            pl.BlockSpec((BN_EP, H2 // 2), lambda i: (i, 0)),
            pl.BlockSpec((BN_EP, H2 // 2), lambda i: (i, 0)),
            pl.BlockSpec((BN_EP, H2), lambda i: (i, 0)),
            pl.BlockSpec((BN_EP, 1), lambda i: (i, 0)),
        ],
        out_shape=[
            jax.ShapeDtypeStruct((N, H2 // 2), jnp.float32),
            jax.ShapeDtypeStruct((N, H2 // 2), jnp.float32),
            jax.ShapeDtypeStruct((N, H2), jnp.float32),
            jax.ShapeDtypeStruct((N, 1), jnp.float32),
        ],
    )(accA, accB, deg0, deg1, xr, w2cat, b1r)


def _ep2_body(accA_ref, accB_ref, recip_ref, gr_ref, b2_ref, out_ref):
    summed = jnp.concatenate([accA_ref[...], accB_ref[...]], axis=1)
    h2 = summed * recip_ref[...] + b2_ref[...] + gr_ref[...]
    y = jnp.maximum(h2, 0.0)
    m = jnp.max(y, axis=1, keepdims=True)
    lse = jnp.log(jnp.sum(jnp.exp(y - m), axis=1, keepdims=True)) + m
    out_ref[...] = y - lse


def _ep2(accA, accB, recip, gr, b2r):
    return pl.pallas_call(
        _ep2_body,
        grid=(N // BN_EP,),
        in_specs=[
            pl.BlockSpec((BN_EP, H2 // 2), lambda i: (i, 0)),
            pl.BlockSpec((BN_EP, H2 // 2), lambda i: (i, 0)),
            pl.BlockSpec((BN_EP, 1), lambda i: (i, 0)),
            pl.BlockSpec((BN_EP, H2), lambda i: (i, 0)),
            pl.BlockSpec((1, H2), lambda i: (0, 0)),
        ],
        out_specs=pl.BlockSpec((BN_EP, H2), lambda i: (i, 0)),
        out_shape=jax.ShapeDtypeStruct((N, H2), jnp.float32),
    )(accA, accB, recip, gr, b2r)


def kernel(x, edge_index, Wl1, Wr1, b1, Wl2, Wr2, b2):
    extra = EPAD - E
    src = jnp.concatenate(
        [edge_index[0], jnp.zeros((extra,), jnp.int32)]).reshape(ECH, CH)
    dst = jnp.concatenate(
        [edge_index[1], jnp.full((extra,), N, jnp.int32)]).reshape(ECH, CH)
    w1cat = jnp.concatenate([Wl1, Wr1], axis=1)
    w2cat = jnp.concatenate([Wl2, Wr2], axis=1)
    b1r = b1.reshape(1, H1)
    b2r = b2.reshape(1, H2)

    hA, hB, xr = _mm1(x.T, w1cat)
    deg0, deg1 = _deg_kernel(dst)
    accA, accB = _agg1(hA, hB, src, dst)
    gA, gB, gr, recip = _ep1(accA, accB, deg0, deg1, xr, w2cat, b1r)
    acc2A, acc2B = _agg2(gA, gB, src, dst)
    return _ep2(acc2A, acc2B, recip, gr, b2r)
